# Initial kernel scaffold; baseline (speedup 1.0000x reference)
import functools

import jax
import jax.numpy as jnp
from jax.experimental import pallas as pl
from jax.experimental.pallas import tpu as pltpu

B = 16384
IN = 512
OUT = 1024
K = 16
BLK = 256

_NEG_INF = jnp.float32(-jnp.inf)


def _mlp3(x, W1, b1, W2, b2, W3, b3):
    h = jnp.maximum(jnp.dot(x, W1, preferred_element_type=jnp.float32) + b1, 0.0)
    h = jnp.maximum(jnp.dot(h, W2, preferred_element_type=jnp.float32) + b2, 0.0)
    return jnp.dot(h, W3, preferred_element_type=jnp.float32) + b3


def _topk_gather(scores, est, iota):
    s = scores
    cols = []
    for _ in range(K):
        m = jnp.max(s, axis=1, keepdims=True)
        cand = jnp.where(s == m, iota, OUT)
        j = jnp.min(cand, axis=1, keepdims=True)
        sel = iota == j
        cols.append(jnp.sum(jnp.where(sel, est, 0.0), axis=1, keepdims=True))
        s = jnp.where(sel, _NEG_INF, s)
    return jnp.concatenate(cols, axis=1)


def _body(x_ref, est_ref,
          roW1, rob1, roW2, rob2, roW3, rob3,
          ruW1, rub1, ruW2, rub2, ruW3, rub3,
          wW1x, wW1e, wb1, wW2, wb2, wW3, wb3,
          so_ref, su_ref, logd_ref):
    x = x_ref[...]
    est = est_ref[...]
    so = _mlp3(x, roW1[...], rob1[...], roW2[...], rob2[...], roW3[...], rob3[...])
    su = _mlp3(x, ruW1[...], rub1[...], ruW2[...], rub2[...], ruW3[...], rub3[...])
    so_ref[...] = so
    su_ref[...] = su

    iota = jax.lax.broadcasted_iota(jnp.int32, (BLK, OUT), 1)
    e_over = _topk_gather(so, est, iota)
    e_under = _topk_gather(su, est, iota)
    e = jnp.concatenate([e_over, e_under], axis=1)

    h = jnp.dot(x, wW1x[...], preferred_element_type=jnp.float32)
    h = h + jnp.dot(e, wW1e[...], preferred_element_type=jnp.float32) + wb1[...]
    h = jnp.maximum(h, 0.0)
    h = jnp.maximum(jnp.dot(h, wW2[...], preferred_element_type=jnp.float32) + wb2[...], 0.0)
    logits = jnp.dot(h, wW3[...], preferred_element_type=jnp.float32) + wb3[...]
    logits = logits - jnp.max(logits, axis=1, keepdims=True)
    p = jnp.exp(logits)
    w = p / jnp.sum(p, axis=1, keepdims=True)
    logd_ref[...] = jnp.sum(e * w, axis=1, keepdims=True)


@jax.jit
def _run(x, estimated_logd, *params):
    grid = (B // BLK,)
    row_spec = lambda nc: pl.BlockSpec((BLK, nc), lambda i: (i, 0))
    full = lambda a: pl.BlockSpec(a.shape, lambda i: (0,) * a.ndim)
    in_specs = [row_spec(IN), row_spec(OUT)] + [full(p) for p in params]
    out_specs = [row_spec(OUT), row_spec(OUT), pl.BlockSpec((BLK, 1), lambda i: (i, 0))]
    out_shape = [
        jax.ShapeDtypeStruct((B, OUT), jnp.float32),
        jax.ShapeDtypeStruct((B, OUT), jnp.float32),
        jax.ShapeDtypeStruct((B, 1), jnp.float32),
    ]
    return pl.pallas_call(
        _body,
        grid=grid,
        in_specs=in_specs,
        out_specs=out_specs,
        out_shape=out_shape,
    )(x, estimated_logd, *params)


def kernel(x, estimated_logd, ro_W1, ro_b1, ro_W2, ro_b2, ro_W3, ro_b3,
           ru_W1, ru_b1, ru_W2, ru_b2, ru_W3, ru_b3,
           w_W1, w_b1, w_W2, w_b2, w_W3, w_b3):
    r2 = lambda b: b.reshape(1, -1)
    so, su, logd = _run(
        x, estimated_logd,
        ro_W1, r2(ro_b1), ro_W2, r2(ro_b2), ro_W3, r2(ro_b3),
        ru_W1, r2(ru_b1), ru_W2, r2(ru_b2), ru_W3, r2(ru_b3),
        w_W1[:IN], w_W1[IN:], r2(w_b1), w_W2, r2(w_b2), w_W3, r2(w_b3),
    )
    return (so, su, logd.reshape(B))


# speedup vs baseline: 5.8402x; 5.8402x over previous
import functools

import jax
import jax.numpy as jnp
from jax.experimental import pallas as pl
from jax.experimental.pallas import tpu as pltpu

B = 16384
IN = 512
OUT = 1024
K = 16
BLK = 256

_NEG_INF = float("-inf")


def _mlp3(x, W1, b1, W2, b2, W3, b3):
    h = jnp.maximum(jnp.dot(x, W1, preferred_element_type=jnp.float32) + b1, 0.0)
    h = jnp.maximum(jnp.dot(h, W2, preferred_element_type=jnp.float32) + b2, 0.0)
    return jnp.dot(h, W3, preferred_element_type=jnp.float32) + b3


def _topk_gather(scores, est, iota):
    s = scores
    cols = []
    for _ in range(K):
        m = jnp.max(s, axis=1, keepdims=True)
        cand = jnp.where(s == m, iota, OUT)
        j = jnp.min(cand, axis=1, keepdims=True)
        sel = iota == j
        cols.append(jnp.sum(jnp.where(sel, est, 0.0), axis=1, keepdims=True))
        s = jnp.where(sel, _NEG_INF, s)
    return jnp.concatenate(cols, axis=1)


def _body(x_ref, est_ref,
          roW1, rob1, roW2, rob2, roW3, rob3,
          ruW1, rub1, ruW2, rub2, ruW3, rub3,
          wW1x, wW1e, wb1, wW2, wb2, wW3, wb3,
          so_ref, su_ref, logd_ref):
    x = x_ref[...]
    est = est_ref[...]
    so = _mlp3(x, roW1[...], rob1[...], roW2[...], rob2[...], roW3[...], rob3[...])
    su = _mlp3(x, ruW1[...], rub1[...], ruW2[...], rub2[...], ruW3[...], rub3[...])
    so_ref[...] = so
    su_ref[...] = su

    iota = jax.lax.broadcasted_iota(jnp.int32, (BLK, OUT), 1)
    e_over = _topk_gather(so, est, iota)
    e_under = _topk_gather(su, est, iota)
    e = jnp.concatenate([e_over, e_under], axis=1)

    h = jnp.dot(x, wW1x[...], preferred_element_type=jnp.float32)
    h = h + jnp.dot(e, wW1e[...], preferred_element_type=jnp.float32) + wb1[...]
    h = jnp.maximum(h, 0.0)
    h = jnp.maximum(jnp.dot(h, wW2[...], preferred_element_type=jnp.float32) + wb2[...], 0.0)
    logits = jnp.dot(h, wW3[...], preferred_element_type=jnp.float32) + wb3[...]
    logits = logits - jnp.max(logits, axis=1, keepdims=True)
    p = jnp.exp(logits)
    w = p / jnp.sum(p, axis=1, keepdims=True)
    logd_ref[...] = jnp.sum(e * w, axis=1, keepdims=True)


@jax.jit
def _run(x, estimated_logd, *params):
    grid = (B // BLK,)
    row_spec = lambda nc: pl.BlockSpec((BLK, nc), lambda i: (i, 0))
    full = lambda a: pl.BlockSpec(a.shape, lambda i: (0,) * a.ndim)
    in_specs = [row_spec(IN), row_spec(OUT)] + [full(p) for p in params]
    out_specs = [row_spec(OUT), row_spec(OUT), pl.BlockSpec((BLK, 1), lambda i: (i, 0))]
    out_shape = [
        jax.ShapeDtypeStruct((B, OUT), jnp.float32),
        jax.ShapeDtypeStruct((B, OUT), jnp.float32),
        jax.ShapeDtypeStruct((B, 1), jnp.float32),
    ]
    return pl.pallas_call(
        _body,
        grid=grid,
        in_specs=in_specs,
        out_specs=out_specs,
        out_shape=out_shape,
    )(x, estimated_logd, *params)


def kernel(x, estimated_logd, ro_W1, ro_b1, ro_W2, ro_b2, ro_W3, ro_b3,
           ru_W1, ru_b1, ru_W2, ru_b2, ru_W3, ru_b3,
           w_W1, w_b1, w_W2, w_b2, w_W3, w_b3):
    r2 = lambda b: b.reshape(1, -1)
    so, su, logd = _run(
        x, estimated_logd,
        ro_W1, r2(ro_b1), ro_W2, r2(ro_b2), ro_W3, r2(ro_b3),
        ru_W1, r2(ru_b1), ru_W2, r2(ru_b2), ru_W3, r2(ru_b3),
        w_W1[:IN], w_W1[IN:], r2(w_b1), w_W2, r2(w_b2), w_W3, r2(w_b3),
    )
    return (so, su, logd.reshape(B))


# speedup vs baseline: 7.1449x; 1.2234x over previous
import functools

import jax
import jax.numpy as jnp
from jax.experimental import pallas as pl
from jax.experimental.pallas import tpu as pltpu

B = 16384
IN = 512
OUT = 1024
K = 16
BLK = 256

_NEG_INF = float("-inf")


def _mlp3(x, W1, b1, W2, b2, W3, b3):
    h = jnp.maximum(jnp.dot(x, W1, preferred_element_type=jnp.float32) + b1, 0.0)
    h = jnp.maximum(jnp.dot(h, W2, preferred_element_type=jnp.float32) + b2, 0.0)
    return jnp.dot(h, W3, preferred_element_type=jnp.float32) + b3


def _topk_gather(scores, est, iota_f):
    s = scores
    cols = []
    for _ in range(K):
        m = jnp.max(s, axis=1, keepdims=True)
        cand = jnp.where(s == m, iota_f, 2048.0)
        j = jnp.min(cand, axis=1, keepdims=True)
        sel = cand == j
        cols.append(jnp.sum(jnp.where(sel, est, 0.0), axis=1, keepdims=True))
        s = jnp.where(sel, _NEG_INF, s)
    return jnp.concatenate(cols, axis=1)


def _body(x_ref, est_ref,
          roW1, rob1, roW2, rob2, roW3, rob3,
          ruW1, rub1, ruW2, rub2, ruW3, rub3,
          wW1x, wW1e, wb1, wW2, wb2, wW3, wb3,
          so_ref, su_ref, logd_ref):
    x = x_ref[...]
    est = est_ref[...]
    so = _mlp3(x, roW1[...], rob1[...], roW2[...], rob2[...], roW3[...], rob3[...])
    su = _mlp3(x, ruW1[...], rub1[...], ruW2[...], rub2[...], ruW3[...], rub3[...])
    so_ref[...] = so
    su_ref[...] = su

    iota_f = jax.lax.broadcasted_iota(jnp.int32, (BLK, OUT), 1).astype(jnp.float32)
    e_over = _topk_gather(so, est, iota_f)
    e_under = _topk_gather(su, est, iota_f)
    e = jnp.concatenate([e_over, e_under], axis=1)

    h = jnp.dot(x, wW1x[...], preferred_element_type=jnp.float32)
    h = h + jnp.dot(e, wW1e[...], preferred_element_type=jnp.float32) + wb1[...]
    h = jnp.maximum(h, 0.0)
    h = jnp.maximum(jnp.dot(h, wW2[...], preferred_element_type=jnp.float32) + wb2[...], 0.0)
    logits = jnp.dot(h, wW3[...], preferred_element_type=jnp.float32) + wb3[...]
    logits = logits - jnp.max(logits, axis=1, keepdims=True)
    p = jnp.exp(logits)
    w = p / jnp.sum(p, axis=1, keepdims=True)
    logd_ref[...] = jnp.sum(e * w, axis=1, keepdims=True)


@jax.jit
def _run(x, estimated_logd, *params):
    grid = (B // BLK,)
    row_spec = lambda nc: pl.BlockSpec((BLK, nc), lambda i: (i, 0))
    full = lambda a: pl.BlockSpec(a.shape, lambda i: (0,) * a.ndim)
    in_specs = [row_spec(IN), row_spec(OUT)] + [full(p) for p in params]
    out_specs = [row_spec(OUT), row_spec(OUT), pl.BlockSpec((BLK, 1), lambda i: (i, 0))]
    out_shape = [
        jax.ShapeDtypeStruct((B, OUT), jnp.float32),
        jax.ShapeDtypeStruct((B, OUT), jnp.float32),
        jax.ShapeDtypeStruct((B, 1), jnp.float32),
    ]
    return pl.pallas_call(
        _body,
        grid=grid,
        in_specs=in_specs,
        out_specs=out_specs,
        out_shape=out_shape,
    )(x, estimated_logd, *params)


def kernel(x, estimated_logd, ro_W1, ro_b1, ro_W2, ro_b2, ro_W3, ro_b3,
           ru_W1, ru_b1, ru_W2, ru_b2, ru_W3, ru_b3,
           w_W1, w_b1, w_W2, w_b2, w_W3, w_b3):
    r2 = lambda b: b.reshape(1, -1)
    so, su, logd = _run(
        x, estimated_logd,
        ro_W1, r2(ro_b1), ro_W2, r2(ro_b2), ro_W3, r2(ro_b3),
        ru_W1, r2(ru_b1), ru_W2, r2(ru_b2), ru_W3, r2(ru_b3),
        w_W1[:IN], w_W1[IN:], r2(w_b1), w_W2, r2(w_b2), w_W3, r2(w_b3),
    )
    return (so, su, logd.reshape(B))
